# Initial kernel scaffold; baseline (speedup 1.0000x reference)
import jax
import jax.numpy as jnp
from jax.experimental import pallas as pl


def kernel(x, x_mark, x_mask, y, y_mark, y_mask, channel_embedding, time_encoder, value_encoder, prop_layers, query_kernel, value_mlp, regression_head):
    raise NotImplementedError("write your pallas kernel here")


# speedup vs baseline: 6.1018x; 6.1018x over previous
import functools

import jax
import jax.numpy as jnp
from jax.experimental import pallas as pl
from jax.experimental.pallas import tpu as pltpu

_B, _L, _N = 8, 96, 21
_S = 24
_DT = 16
_KNN = 16
_BIG = 3.0e38


def _mlp(params, x):
    for i, (W, b) in enumerate(params):
        x = x @ W + b
        if i < len(params) - 1:
            x = jax.nn.relu(x)
    return x


def _dotx(a, b):
    return jax.lax.dot(a, b, precision=jax.lax.Precision.HIGHEST)


def _knn_body(eq_ref, eb_ref, ebt_ref, dc_ref, out_ref, *, rows, cand_t):
    b = pl.program_id(0)
    eq = eq_ref[...]
    eb = eb_ref[0]
    ebt = ebt_ref[0]
    dc = dc_ref[...]
    del eb

    d = jnp.zeros((rows, _L), jnp.float32)
    for f in range(_DT):
        diff = eq[:, f:f + 1] - ebt[f:f + 1, :]
        d = d + diff * diff

    iot = jax.lax.broadcasted_iota(jnp.int32, (rows, _L), 1).astype(jnp.float32)
    vals = d
    vcols, icols = [], []
    for _ in range(cand_t):
        m = jnp.min(vals, axis=1, keepdims=True)
        pick = jnp.min(jnp.where(vals <= m, iot, 1e9), axis=1, keepdims=True)
        vcols.append(m)
        icols.append(pick)
        vals = jnp.where(iot == pick, _BIG, vals)
    dtv = jnp.concatenate(vcols, axis=1)
    dti = jnp.concatenate(icols, axis=1)

    R = rows * _N
    C = cand_t * _N
    r0 = jax.lax.broadcasted_iota(jnp.int32, (R, rows), 0)
    r1 = jax.lax.broadcasted_iota(jnp.int32, (R, rows), 1)
    orow = (r0 // _N == r1).astype(jnp.float32)
    a_val = _dotx(orow, dtv)
    a_idx = _dotx(orow, dti)
    t0 = jax.lax.broadcasted_iota(jnp.int32, (R, _N), 0)
    t1 = jax.lax.broadcasted_iota(jnp.int32, (R, _N), 1)
    tvar = (t0 % _N == t1).astype(jnp.float32)
    c_val = _dotx(tvar, dc)
    p0 = jax.lax.broadcasted_iota(jnp.int32, (cand_t, C), 0)
    p1 = jax.lax.broadcasted_iota(jnp.int32, (cand_t, C), 1)
    pmat = (p1 // _N == p0).astype(jnp.float32)
    q0 = jax.lax.broadcasted_iota(jnp.int32, (_N, C), 0)
    q1 = jax.lax.broadcasted_iota(jnp.int32, (_N, C), 1)
    qmat = (q1 % _N == q0).astype(jnp.float32)
    cand = _dotx(a_val, pmat) + _dotx(c_val, qmat)
    cti = _dotx(a_idx, pmat)
    cvj = (jax.lax.broadcasted_iota(jnp.int32, (R, C), 1) % _N).astype(jnp.float32)
    base_bt = (b * _L) * 1.0
    fidx = (base_bt + cti) * _N + cvj

    iotc = jax.lax.broadcasted_iota(jnp.int32, (R, C), 1).astype(jnp.float32)
    vals = cand
    cols = []
    for _ in range(_KNN):
        m = jnp.min(vals, axis=1, keepdims=True)
        pick = jnp.min(jnp.where(vals <= m, iotc, 1e9), axis=1, keepdims=True)
        hit = iotc == pick
        cols.append(jnp.sum(jnp.where(hit, fidx, 0.0), axis=1, keepdims=True))
        vals = jnp.where(hit, _BIG, vals)
    out_ref[...] = jnp.concatenate(cols, axis=1).astype(jnp.int32)


def _knn_hist(e_t, dc):
    e_b = e_t.reshape(_B, _L, _DT)
    e_bt = jnp.transpose(e_b, (0, 2, 1))
    blocks_per_b = _L // 16
    grid = (_B,)
    return pl.pallas_call(
        functools.partial(_knn_body, rows=_L, cand_t=16),
        grid=grid,
        in_specs=[
            pl.BlockSpec((_L, _DT), lambda b: (b, 0)),
            pl.BlockSpec((1, _L, _DT), lambda b: (b, 0, 0)),
            pl.BlockSpec((1, _DT, _L), lambda b: (b, 0, 0)),
            pl.BlockSpec((_N, _N), lambda b: (0, 0)),
        ],
        out_specs=pl.BlockSpec((_L * _N, _KNN), lambda b: (b, 0)),
        out_shape=jax.ShapeDtypeStruct((_B * _L * _N, _KNN), jnp.int32),
    )(e_t, e_b, e_bt, dc)


def _knn_query(e_tq, e_t, dc):
    e_b = e_t.reshape(_B, _L, _DT)
    e_bt = jnp.transpose(e_b, (0, 2, 1))
    return pl.pallas_call(
        functools.partial(_knn_body, rows=_S, cand_t=16),
        grid=(_B,),
        in_specs=[
            pl.BlockSpec((_S, _DT), lambda b: (b, 0)),
            pl.BlockSpec((1, _L, _DT), lambda b: (b, 0, 0)),
            pl.BlockSpec((1, _DT, _L), lambda b: (b, 0, 0)),
            pl.BlockSpec((_N, _N), lambda b: (0, 0)),
        ],
        out_specs=pl.BlockSpec((_S * _N, _KNN), lambda b: (b, 0)),
        out_shape=jax.ShapeDtypeStruct((_B * _S * _N, _KNN), jnp.int32),
    )(e_tq, e_b, e_bt, dc)


def _layer_norm(x, gamma, beta, eps=1e-5):
    mu = jnp.mean(x, axis=-1, keepdims=True)
    var = jnp.mean((x - mu) ** 2, axis=-1, keepdims=True)
    return (x - mu) / jnp.sqrt(var + eps) * gamma + beta


def kernel(x, x_mark, x_mask, y, y_mark, y_mask, channel_embedding, time_encoder,
           value_encoder, prop_layers, query_kernel, value_mlp, regression_head):
    times_bt = x_mark[:, :, 0].reshape(-1, 1)
    e_t = _mlp(time_encoder, times_bt)
    e_c = channel_embedding
    p = jnp.concatenate([
        jnp.broadcast_to(e_c[None, :, :], (_B * _L, _N, _DT)),
        jnp.broadcast_to(e_t[:, None, :], (_B * _L, _N, _DT)),
    ], axis=-1).reshape(-1, 2 * _DT)
    h = _mlp(value_encoder, x.reshape(-1, 1))
    t_pt = jnp.broadcast_to(times_bt[:, None, :], (_B * _L, _N, 1)).reshape(-1)

    dc = jnp.sum((e_c[:, None, :] - e_c[None, :, :]) ** 2, axis=-1)
    nbr = _knn_hist(e_t, dc)

    for lp in prop_layers:
        Wr1, b1r = lp['relation_kernel'][0]
        w2r, b2r = lp['relation_kernel'][1]
        Wm1, b1m = lp['message_mlp'][0]
        Wm2, b2m = lp['message_mlp'][1]
        A, Bh, Ch = Wr1[:32], Wr1[32:96], Wr1[96:]
        Cm, Am = Wm1[:64], Wm1[64:]
        pa = p @ A
        U = pa + h @ Bh + b1r
        V = h @ Ch - pa
        pam = p @ Am
        Uu = pam + b1m
        Vv = h @ Cm - pam
        Vj = V[nbr]
        Vvj = Vv[nbr]
        s = jax.nn.relu(U[:, None, :] + Vj) @ w2r[:, 0] + b2r[0]
        tj = t_pt[nbr]
        s = jnp.where(tj > t_pt[:, None], -jnp.inf, s)
        attn = jax.nn.softmax(s, axis=-1)
        acc = jnp.sum(attn[..., None] * jax.nn.relu(Uu[:, None, :] + Vvj), axis=1)
        agg = acc @ Wm2 + b2m
        h = _layer_norm(h + _mlp(lp['update_mlp'], agg), lp['ln_gamma'], lp['ln_beta'])

    q_times = y_mark[:, :, 0].reshape(-1, 1)
    e_tq = _mlp(time_encoder, q_times)
    p_q = jnp.concatenate([
        jnp.broadcast_to(e_c[None, :, :], (_B * _S, _N, _DT)),
        jnp.broadcast_to(e_tq[:, None, :], (_B * _S, _N, _DT)),
    ], axis=-1).reshape(-1, 2 * _DT)
    q_nbr = _knn_query(e_tq, e_t, dc)

    Wq1, b1q = query_kernel[0]
    w2q, b2q = query_kernel[1]
    Aq, Cq = Wq1[:32], Wq1[32:]
    Uq = p_q @ Aq + b1q
    Vq = h @ Cq - p @ Aq
    ht = _mlp(value_mlp, h)
    Vqj = Vq[q_nbr]
    htj = ht[q_nbr]
    sq = jax.nn.relu(Uq[:, None, :] + Vqj) @ w2q[:, 0] + b2q[0]
    q_attn = jax.nn.softmax(sq, axis=-1)
    h_q = jnp.sum(q_attn[..., None] * htj, axis=1)
    preds = _mlp(regression_head, h_q)
    outputs = preds.reshape(_B, _S, _N)
    return (outputs, y, y_mask)
